# Initial kernel scaffold; baseline (speedup 1.0000x reference)
import jax
import jax.numpy as jnp
from jax.experimental import pallas as pl


def kernel(z, edge_index):
    raise NotImplementedError("write your pallas kernel here")


# speedup vs baseline: 4.9083x; 4.9083x over previous
import jax
import jax.numpy as jnp
from jax import lax
from jax.experimental import pallas as pl
from jax.experimental.pallas import tpu as pltpu
from jax.experimental.pallas import tpu_sc as plsc

D = 128
L = 16
NC, NS = 2, 16
NW = NC * NS
C = 400


def _decoder_body(z_hbm, src_hbm, dst_hbm, out_hbm,
                  sidx_v, didx_v, srow_v, drow_v, m_v, out_v, sem):
    num_edges = src_hbm.shape[0]
    ew = num_edges // NW
    nch = ew // C
    ng = C // L

    wid = lax.axis_index("s") * NC + lax.axis_index("c")
    base_w = wid * ew
    lane = lax.iota(jnp.int32, L)
    scat_base = lane * L

    def chunk_body(ch, carry):
        base = base_w + ch * C
        pltpu.sync_copy(src_hbm.at[pl.ds(base, C)], sidx_v)
        pltpu.sync_copy(dst_hbm.at[pl.ds(base, C)], didx_v)
        cp_s = pltpu.async_copy(z_hbm.at[sidx_v], srow_v, sem)
        cp_d = pltpu.async_copy(z_hbm.at[didx_v], drow_v, sem)
        cp_s.wait()
        cp_d.wait()

        def group_body(g, gcarry):
            rowbase = g * L
            for e in range(L):
                row = rowbase + e
                acc = srow_v[row, pl.ds(0, L)] * drow_v[row, pl.ds(0, L)]
                for c in range(1, D // L):
                    acc = acc + (srow_v[row, pl.ds(c * L, L)]
                                 * drow_v[row, pl.ds(c * L, L)])
                plsc.store_scatter(m_v, [scat_base + e], acc)
            tot = m_v[pl.ds(0, L)]
            for l in range(1, L):
                tot = tot + m_v[pl.ds(l * L, L)]
            out_v[pl.ds(rowbase, L)] = 1.0 / (1.0 + jnp.exp(-tot))
            return gcarry

        lax.fori_loop(0, ng, group_body, 0)
        pltpu.sync_copy(out_v, out_hbm.at[pl.ds(base, C)])
        return carry

    lax.fori_loop(0, nch, chunk_body, 0)


def kernel(z, edge_index):
    num_edges = edge_index.shape[1]
    ei = edge_index.astype(jnp.int32)
    src, dst = ei[0], ei[1]
    mesh = plsc.VectorSubcoreMesh(core_axis_name="c", subcore_axis_name="s",
                                  num_cores=NC, num_subcores=NS)
    k = pl.kernel(
        _decoder_body,
        out_type=jax.ShapeDtypeStruct((num_edges,), jnp.float32),
        mesh=mesh,
        compiler_params=pltpu.CompilerParams(needs_layout_passes=False),
        scratch_types=[
            pltpu.VMEM((C,), jnp.int32),
            pltpu.VMEM((C,), jnp.int32),
            pltpu.VMEM((C, D), jnp.float32),
            pltpu.VMEM((C, D), jnp.float32),
            pltpu.VMEM((L * L,), jnp.float32),
            pltpu.VMEM((C,), jnp.float32),
            pltpu.SemaphoreType.DMA,
        ],
    )
    return k(z, src, dst)


# speedup vs baseline: 7.7580x; 1.5806x over previous
import jax
import jax.numpy as jnp
from jax import lax
from jax.experimental import pallas as pl
from jax.experimental.pallas import tpu as pltpu
from jax.experimental.pallas import tpu_sc as plsc

D = 128
L = 16
NC, NS = 2, 16
NW = NC * NS
C = 80
NG = C // L


def _decoder_body(z_hbm, src_hbm, dst_hbm, out_hbm,
                  sidx0, sidx1, didx0, didx1,
                  srow0, srow1, drow0, drow1,
                  m_v, out_v,
                  sem_r0, sem_r1, sem_i0, sem_i1):
    num_edges = src_hbm.shape[0]
    ew = num_edges // NW
    nch = ew // C

    wid = lax.axis_index("s") * NC + lax.axis_index("c")
    base_w = wid * ew
    lane = lax.iota(jnp.int32, L)
    scat_base = lane * L

    def issue_idx(ch, si, di, sem):
        base = base_w + ch * C
        pltpu.async_copy(src_hbm.at[pl.ds(base, C)], si, sem)
        pltpu.async_copy(dst_hbm.at[pl.ds(base, C)], di, sem)

    def wait_idx(si, di, sem):
        pltpu.make_async_copy(src_hbm.at[pl.ds(0, C)], si, sem).wait()
        pltpu.make_async_copy(dst_hbm.at[pl.ds(0, C)], di, sem).wait()

    def issue_gather(si, di, sr, dr, sem):
        pltpu.async_copy(z_hbm.at[si], sr, sem)
        pltpu.async_copy(z_hbm.at[di], dr, sem)

    def wait_gather(si, di, sr, dr, sem):
        pltpu.make_async_copy(z_hbm.at[si], sr, sem).wait()
        pltpu.make_async_copy(z_hbm.at[di], dr, sem).wait()

    def compute(ch, sr, dr):
        out_base = ch * C

        def group_body(g, gcarry):
            rowbase = g * L
            for e in range(L):
                row = rowbase + e
                acc = sr[row, pl.ds(0, L)] * dr[row, pl.ds(0, L)]
                for c in range(1, D // L):
                    acc = acc + (sr[row, pl.ds(c * L, L)]
                                 * dr[row, pl.ds(c * L, L)])
                plsc.store_scatter(m_v, [scat_base + e], acc)
            tot = m_v[pl.ds(0, L)]
            for l in range(1, L):
                tot = tot + m_v[pl.ds(l * L, L)]
            out_v[pl.ds(out_base + rowbase, L)] = 1.0 / (1.0 + jnp.exp(-tot))
            return gcarry

        lax.fori_loop(0, NG, group_body, 0)

    pltpu.sync_copy(src_hbm.at[pl.ds(base_w, C)], sidx0)
    pltpu.sync_copy(dst_hbm.at[pl.ds(base_w, C)], didx0)
    issue_gather(sidx0, didx0, srow0, drow0, sem_r0)
    issue_idx(1, sidx1, didx1, sem_i1)
    wait_idx(sidx1, didx1, sem_i1)
    issue_gather(sidx1, didx1, srow1, drow1, sem_r1)
    wait_gather(sidx0, didx0, srow0, drow0, sem_r0)
    issue_idx(2, sidx0, didx0, sem_i0)
    compute(0, srow0, drow0)

    def pair_body(i, carry):
        c1 = 2 * i + 1
        c2 = 2 * i + 2
        wait_idx(sidx0, didx0, sem_i0)
        issue_gather(sidx0, didx0, srow0, drow0, sem_r0)
        wait_gather(sidx1, didx1, srow1, drow1, sem_r1)
        issue_idx(c1 + 2, sidx1, didx1, sem_i1)
        compute(c1, srow1, drow1)
        wait_idx(sidx1, didx1, sem_i1)
        issue_gather(sidx1, didx1, srow1, drow1, sem_r1)
        wait_gather(sidx0, didx0, srow0, drow0, sem_r0)
        issue_idx(c2 + 2, sidx0, didx0, sem_i0)
        compute(c2, srow0, drow0)
        return carry

    lax.fori_loop(0, (nch - 3) // 2, pair_body, 0)

    wait_idx(sidx0, didx0, sem_i0)
    issue_gather(sidx0, didx0, srow0, drow0, sem_r0)
    wait_gather(sidx1, didx1, srow1, drow1, sem_r1)
    compute(nch - 2, srow1, drow1)
    wait_gather(sidx0, didx0, srow0, drow0, sem_r0)
    compute(nch - 1, srow0, drow0)

    pltpu.sync_copy(out_v, out_hbm.at[pl.ds(base_w, ew)])


def kernel(z, edge_index):
    num_edges = edge_index.shape[1]
    ei = edge_index.astype(jnp.int32)
    src, dst = ei[0], ei[1]
    ew = num_edges // NW
    mesh = plsc.VectorSubcoreMesh(core_axis_name="c", subcore_axis_name="s",
                                  num_cores=NC, num_subcores=NS)
    k = pl.kernel(
        _decoder_body,
        out_type=jax.ShapeDtypeStruct((num_edges,), jnp.float32),
        mesh=mesh,
        compiler_params=pltpu.CompilerParams(needs_layout_passes=False),
        scratch_types=[
            pltpu.VMEM((C,), jnp.int32),
            pltpu.VMEM((C,), jnp.int32),
            pltpu.VMEM((C,), jnp.int32),
            pltpu.VMEM((C,), jnp.int32),
            pltpu.VMEM((C, D), jnp.float32),
            pltpu.VMEM((C, D), jnp.float32),
            pltpu.VMEM((C, D), jnp.float32),
            pltpu.VMEM((C, D), jnp.float32),
            pltpu.VMEM((L * L,), jnp.float32),
            pltpu.VMEM((ew,), jnp.float32),
            pltpu.SemaphoreType.DMA,
            pltpu.SemaphoreType.DMA,
            pltpu.SemaphoreType.DMA,
            pltpu.SemaphoreType.DMA,
        ],
    )
    return k(z, src, dst)
